# baseline (speedup 1.0000x reference)
import functools

import jax
import jax.numpy as jnp
from jax import lax
from jax.experimental import pallas as pl
from jax.experimental.pallas import tpu as pltpu
from jax.experimental.pallas import tpu_sc as plsc

VOCAB = 100000
EMBED = 64
BATCH = 1024


@functools.lru_cache(maxsize=None)
def _make_sc_gather_t(V, D, B):
    info = plsc.get_sparse_core_info()
    NC, NS = info.num_cores, info.num_subcores
    NW = NC * NS
    assert D % NW == 0 and B % 16 == 0
    rows_per_w = D // NW
    mesh = plsc.VectorSubcoreMesh(core_axis_name="c", subcore_axis_name="s")

    @functools.partial(
        pl.kernel,
        mesh=mesh,
        out_type=jax.ShapeDtypeStruct((D, B), jnp.float32),
        scratch_types=[
            pltpu.VMEM((B,), jnp.int32),
            pltpu.VMEM((V,), jnp.float32),
            pltpu.VMEM((B,), jnp.float32),
        ],
        compiler_params=pltpu.CompilerParams(needs_layout_passes=False),
    )
    def gather(table_hbm, idx_hbm, out_hbm, idx_v, row_v, out_v):
        wid = lax.axis_index("s") * NC + lax.axis_index("c")
        pltpu.sync_copy(idx_hbm, idx_v)
        idx_vecs = [idx_v[pl.ds(16 * j, 16)] for j in range(B // 16)]
        for r in range(rows_per_w):
            e = wid * rows_per_w + r
            pltpu.sync_copy(table_hbm.at[e], row_v)
            for j in range(B // 16):
                out_v[pl.ds(16 * j, 16)] = plsc.load_gather(row_v, [idx_vecs[j]])
            pltpu.sync_copy(out_v, out_hbm.at[e])

    return gather


BN = 1024


def _proj_body(wt_ref, b_ref, embt_ref, outt_ref):
    b_row = b_ref[...].reshape(1, b_ref.shape[0])
    wb = jnp.concatenate([wt_ref[...], b_row], axis=0)
    ea = jnp.concatenate(
        [embt_ref[...], jnp.ones((1, embt_ref.shape[1]), jnp.float32)], axis=0
    )
    outt_ref[...] = lax.dot_general(
        wb, ea, (((0,), (0,)), ((), ())), preferred_element_type=jnp.float32
    )


@functools.lru_cache(maxsize=None)
def _make_proj_t(V, D, B):
    grid = pl.cdiv(V, BN)
    return pl.pallas_call(
        _proj_body,
        grid=(grid,),
        in_specs=[
            pl.BlockSpec((D, BN), lambda i: (0, i)),
            pl.BlockSpec((BN,), lambda i: (i,)),
            pl.BlockSpec((D, B), lambda i: (0, 0)),
        ],
        out_specs=pl.BlockSpec((BN, B), lambda i: (i, 0)),
        out_shape=jax.ShapeDtypeStruct((V, B), jnp.float32),
        compiler_params=pltpu.CompilerParams(
            dimension_semantics=("arbitrary",),
        ),
    )


@jax.jit
def kernel(center, emb_table, W, b):
    table_t = emb_table.T
    wt = W.T
    embt = _make_sc_gather_t(VOCAB, EMBED, BATCH)(table_t, center)
    outt = _make_proj_t(VOCAB, EMBED, BATCH)(wt, b, embt)
    return outt.T


# speedup vs baseline: 1.1535x; 1.1535x over previous
import functools

import jax
import jax.numpy as jnp
from jax import lax
from jax.experimental import pallas as pl
from jax.experimental.pallas import tpu as pltpu
from jax.experimental.pallas import tpu_sc as plsc

VOCAB = 100000
EMBED = 64
BATCH = 1024


@functools.lru_cache(maxsize=None)
def _make_sc_gather_t(V, D, B):
    info = plsc.get_sparse_core_info()
    NC, NS = info.num_cores, info.num_subcores
    NW = NC * NS
    assert D % NW == 0 and B % 16 == 0
    rows_per_w = D // NW
    mesh = plsc.VectorSubcoreMesh(core_axis_name="c", subcore_axis_name="s")

    @functools.partial(
        pl.kernel,
        mesh=mesh,
        out_type=jax.ShapeDtypeStruct((D, B), jnp.float32),
        scratch_types=[
            pltpu.VMEM((B,), jnp.int32),
            pltpu.VMEM((V,), jnp.float32),
            pltpu.VMEM((B,), jnp.float32),
        ],
        compiler_params=pltpu.CompilerParams(needs_layout_passes=False),
    )
    def gather(table_hbm, idx_hbm, out_hbm, idx_v, row_v, out_v):
        wid = lax.axis_index("s") * NC + lax.axis_index("c")
        pltpu.sync_copy(idx_hbm, idx_v)
        idx_vecs = [idx_v[pl.ds(16 * j, 16)] for j in range(B // 16)]
        for r in range(rows_per_w):
            e = wid * rows_per_w + r
            pltpu.sync_copy(table_hbm.at[e], row_v)
            for j in range(B // 16):
                out_v[pl.ds(16 * j, 16)] = plsc.load_gather(row_v, [idx_vecs[j]])
            pltpu.sync_copy(out_v, out_hbm.at[e])

    return gather


BN = 4096


def _proj_body(wt_ref, b_ref, embt_ref, outt_ref):
    b_row = b_ref[...].reshape(1, b_ref.shape[0])
    wb = jnp.concatenate([wt_ref[...], b_row], axis=0)
    ea = jnp.concatenate(
        [embt_ref[...], jnp.ones((1, embt_ref.shape[1]), jnp.float32)], axis=0
    )
    outt_ref[...] = lax.dot_general(
        wb, ea, (((0,), (0,)), ((), ())), preferred_element_type=jnp.float32
    )


@functools.lru_cache(maxsize=None)
def _make_proj_t(V, D, B):
    grid = pl.cdiv(V, BN)
    return pl.pallas_call(
        _proj_body,
        grid=(grid,),
        in_specs=[
            pl.BlockSpec((D, BN), lambda i: (0, i)),
            pl.BlockSpec((BN,), lambda i: (i,)),
            pl.BlockSpec((D, B), lambda i: (0, 0)),
        ],
        out_specs=pl.BlockSpec((BN, B), lambda i: (i, 0)),
        out_shape=jax.ShapeDtypeStruct((V, B), jnp.float32),
        compiler_params=pltpu.CompilerParams(
            dimension_semantics=("arbitrary",),
        ),
    )


@jax.jit
def kernel(center, emb_table, W, b):
    table_t = emb_table.T
    wt = W.T
    embt = _make_sc_gather_t(VOCAB, EMBED, BATCH)(table_t, center)
    outt = _make_proj_t(VOCAB, EMBED, BATCH)(wt, b, embt)
    return outt.T
